# Initial kernel scaffold; baseline (speedup 1.0000x reference)
import jax
import jax.numpy as jnp
from jax.experimental import pallas as pl


def kernel(x, W):
    raise NotImplementedError("write your pallas kernel here")


# speedup vs baseline: 1.6193x; 1.6193x over previous
import jax
import jax.numpy as jnp
from jax.experimental import pallas as pl

_K = 64


def _matmul_body(x_ref, w_ref, out_ref):
    out_ref[0] = jax.lax.dot_general(
        w_ref[...], x_ref[0],
        dimension_numbers=(((1,), (1,)), ((), ())),
        preferred_element_type=jnp.float32,
    )


def _topk_body(s_ref, idx_ref, val_ref):
    s = s_ref[0]
    H, T = s.shape
    iota = jax.lax.broadcasted_iota(jnp.int32, (H, T), 1)

    def body(j, s):
        m = jnp.max(s, axis=1, keepdims=True)
        eq = s == m
        idx = jnp.min(jnp.where(eq, iota, T), axis=1, keepdims=True)
        val_ref[0, j, :] = m[:, 0]
        idx_ref[0, j, :] = idx[:, 0]
        return jnp.where(iota == idx, -jnp.inf, s)

    jax.lax.fori_loop(0, _K, body, s)


def kernel(x, W):
    B, T, D = x.shape
    H = W.shape[0]
    Tt = 512

    scores = pl.pallas_call(
        _matmul_body,
        grid=(B, T // Tt),
        in_specs=[
            pl.BlockSpec((1, Tt, D), lambda b, t: (b, t, 0)),
            pl.BlockSpec((H, D), lambda b, t: (0, 0)),
        ],
        out_specs=pl.BlockSpec((1, H, Tt), lambda b, t: (b, 0, t)),
        out_shape=jax.ShapeDtypeStruct((B, H, T), jnp.float32),
    )(x, W)

    idx, val = pl.pallas_call(
        _topk_body,
        grid=(B,),
        in_specs=[pl.BlockSpec((1, H, T), lambda b: (b, 0, 0))],
        out_specs=[
            pl.BlockSpec((1, _K, H), lambda b: (b, 0, 0)),
            pl.BlockSpec((1, _K, H), lambda b: (b, 0, 0)),
        ],
        out_shape=[
            jax.ShapeDtypeStruct((B, _K, H), jnp.int32),
            jax.ShapeDtypeStruct((B, _K, H), jnp.float32),
        ],
    )(scores)

    return jnp.transpose(idx, (0, 2, 1)), jnp.transpose(val, (0, 2, 1))


# speedup vs baseline: 2.7260x; 1.6834x over previous
import functools

import jax
import jax.numpy as jnp
from jax import lax
from jax.experimental import pallas as pl
from jax.experimental.pallas import tpu as pltpu
from jax.experimental.pallas import tpu_sc as plsc

_K = 64
_L = 16
_RPW = 4
_T = 4096
_NCHUNK = _T // _L
_NSUPER = _NCHUNK // _L
_NEG_INF = float("-inf")


def _matmul_body(x_ref, w_ref, out_ref):
    out_ref[0] = jax.lax.dot_general(
        w_ref[...], x_ref[0],
        dimension_numbers=(((1,), (1,)), ((), ())),
        preferred_element_type=jnp.float32,
    )


def _sc_topk_body(scores_hbm, idx_hbm, val_hbm, rows_v, l1_v, oidx_v, oval_v):
    wid = lax.axis_index("s") * 2 + lax.axis_index("c")
    base = wid * _RPW
    pltpu.sync_copy(scores_hbm.at[pl.ds(base * _T, _RPW * _T)], rows_v)

    iota = lax.iota(jnp.int32, _L)

    def build(s, l2s):
        out = []
        for r in range(_RPW):
            acc = jnp.full((_L,), _NEG_INF, jnp.float32)
            for j in range(_L):
                v = rows_v[pl.ds(r * _T + (s * _L + j) * _L, _L)]
                m = lax.reduce_max(v, axes=(0,))
                acc = jnp.where(iota == j, m, acc)
            l1_v[pl.ds(r * _NCHUNK + s * _L, _L)] = acc
            sm = lax.reduce_max(acc, axes=(0,))
            out.append(jnp.where(iota == s, sm, l2s[r]))
        return tuple(out)

    l2s = tuple(jnp.full((_L,), _NEG_INF, jnp.float32) for _ in range(_RPW))
    l2s = lax.fori_loop(0, _NSUPER, build, l2s)

    def extract(k, l2s):
        out = []
        for r in range(_RPW):
            l2 = l2s[r]
            m = lax.reduce_max(l2, axes=(0,))
            s = lax.reduce_min(jnp.where(l2 == m, iota, _L), axes=(0,))
            l1c = l1_v[pl.ds(r * _NCHUNK + s * _L, _L)]
            c = s * _L + lax.reduce_min(jnp.where(l1c == m, iota, _L),
                                        axes=(0,))
            dv = rows_v[pl.ds(r * _T + c * _L, _L)]
            lx = lax.reduce_min(jnp.where(dv == m, iota, _L), axes=(0,))
            kq, kr = k // _L, k % _L
            ob = r * _K + kq * _L
            oi = oidx_v[pl.ds(ob, _L)]
            oidx_v[pl.ds(ob, _L)] = jnp.where(iota == kr, c * _L + lx, oi)
            ov = oval_v[pl.ds(ob, _L)]
            oval_v[pl.ds(ob, _L)] = jnp.where(iota == kr, m, ov)
            dv2 = jnp.where(iota == lx, _NEG_INF, dv)
            rows_v[pl.ds(r * _T + c * _L, _L)] = dv2
            nm = lax.reduce_max(dv2, axes=(0,))
            l1c2 = l1_v[pl.ds(r * _NCHUNK + s * _L, _L)]
            l1c2 = jnp.where(iota == c - s * _L, nm, l1c2)
            l1_v[pl.ds(r * _NCHUNK + s * _L, _L)] = l1c2
            sm = lax.reduce_max(l1c2, axes=(0,))
            out.append(jnp.where(iota == s, sm, l2))
        return tuple(out)

    lax.fori_loop(0, _K, extract, l2s)

    pltpu.sync_copy(oidx_v, idx_hbm.at[pl.ds(base * _K, _RPW * _K)])
    pltpu.sync_copy(oval_v, val_hbm.at[pl.ds(base * _K, _RPW * _K)])


def kernel(x, W):
    B, T, D = x.shape
    H = W.shape[0]
    Tt = 512

    scores = pl.pallas_call(
        _matmul_body,
        grid=(B, T // Tt),
        in_specs=[
            pl.BlockSpec((1, Tt, D), lambda b, t: (b, t, 0)),
            pl.BlockSpec((H, D), lambda b, t: (0, 0)),
        ],
        out_specs=pl.BlockSpec((1, H, Tt), lambda b, t: (b, 0, t)),
        out_shape=jax.ShapeDtypeStruct((B, H, T), jnp.float32),
    )(x, W)

    R = B * H
    mesh = plsc.VectorSubcoreMesh(core_axis_name="c", subcore_axis_name="s")
    topk = functools.partial(
        pl.kernel,
        mesh=mesh,
        out_type=[
            jax.ShapeDtypeStruct((R * _K,), jnp.int32),
            jax.ShapeDtypeStruct((R * _K,), jnp.float32),
        ],
        scratch_types=[
            pltpu.VMEM((_RPW * _T,), jnp.float32),
            pltpu.VMEM((_RPW * _NCHUNK,), jnp.float32),
            pltpu.VMEM((_RPW * _K,), jnp.int32),
            pltpu.VMEM((_RPW * _K,), jnp.float32),
        ],
        compiler_params=pltpu.CompilerParams(
            needs_layout_passes=False, use_tc_tiling_on_sc=False),
    )(_sc_topk_body)

    idx, val = topk(scores.reshape(R * T))
    return idx.reshape(B, H, _K), val.reshape(B, H, _K)


# speedup vs baseline: 2.9354x; 1.0768x over previous
import functools

import jax
import jax.numpy as jnp
from jax import lax
from jax.experimental import pallas as pl
from jax.experimental.pallas import tpu as pltpu
from jax.experimental.pallas import tpu_sc as plsc

_K = 64
_L = 16
_RPW = 4
_T = 4096
_NCHUNK = _T // _L
_NSUPER = _NCHUNK // _L
_NEG_INF = float("-inf")


def _matmul_body(x_ref, w_ref, out_ref):
    out_ref[0] = jax.lax.dot_general(
        w_ref[...], x_ref[0],
        dimension_numbers=(((1,), (1,)), ((), ())),
        preferred_element_type=jnp.float32,
    )


def _sc_topk_body(scores_hbm, idx_hbm, val_hbm, rows_v, l1_v, oidx_v, oval_v):
    wid = lax.axis_index("s") * 2 + lax.axis_index("c")
    base = wid * _RPW
    pltpu.sync_copy(scores_hbm.at[pl.ds(base * _T, _RPW * _T)], rows_v)

    iota = lax.iota(jnp.int32, _L)

    def build(s, l2s):
        out = []
        for r in range(_RPW):
            goff = r * _T + s * (_L * _L) + iota * _L
            acc = plsc.load_gather(rows_v, [goff])
            for j in range(1, _L):
                acc = jnp.maximum(acc, plsc.load_gather(rows_v, [goff + j]))
            l1_v[pl.ds(r * _NCHUNK + s * _L, _L)] = acc
            sm = lax.reduce_max(acc, axes=(0,))
            out.append(jnp.where(iota == s, sm, l2s[r]))
        return tuple(out)

    l2s = tuple(jnp.full((_L,), _NEG_INF, jnp.float32) for _ in range(_RPW))
    l2s = lax.fori_loop(0, _NSUPER, build, l2s)

    def extract(k, l2s):
        out = []
        for r in range(_RPW):
            l2 = l2s[r]
            m = lax.reduce_max(l2, axes=(0,))
            s = plsc.all_reduce_ffs(l2 == m)[0]
            l1c = l1_v[pl.ds(r * _NCHUNK + s * _L, _L)]
            c = s * _L + plsc.all_reduce_ffs(l1c == m)[0]
            dv = rows_v[pl.ds(r * _T + c * _L, _L)]
            lx = plsc.all_reduce_ffs(dv == m)[0]
            kq, kr = k // _L, k % _L
            ob = r * _K + kq * _L
            oi = oidx_v[pl.ds(ob, _L)]
            oidx_v[pl.ds(ob, _L)] = jnp.where(iota == kr, c * _L + lx, oi)
            ov = oval_v[pl.ds(ob, _L)]
            oval_v[pl.ds(ob, _L)] = jnp.where(iota == kr, m, ov)
            dv2 = jnp.where(iota == lx, _NEG_INF, dv)
            rows_v[pl.ds(r * _T + c * _L, _L)] = dv2
            nm = lax.reduce_max(dv2, axes=(0,))
            l1c2 = l1_v[pl.ds(r * _NCHUNK + s * _L, _L)]
            l1c2 = jnp.where(iota == c - s * _L, nm, l1c2)
            l1_v[pl.ds(r * _NCHUNK + s * _L, _L)] = l1c2
            sm = lax.reduce_max(l1c2, axes=(0,))
            out.append(jnp.where(iota == s, sm, l2))
        return tuple(out)

    lax.fori_loop(0, _K, extract, l2s)

    pltpu.sync_copy(oidx_v, idx_hbm.at[pl.ds(base * _K, _RPW * _K)])
    pltpu.sync_copy(oval_v, val_hbm.at[pl.ds(base * _K, _RPW * _K)])


def kernel(x, W):
    B, T, D = x.shape
    H = W.shape[0]
    Tt = 512

    scores = pl.pallas_call(
        _matmul_body,
        grid=(B, T // Tt),
        in_specs=[
            pl.BlockSpec((1, Tt, D), lambda b, t: (b, t, 0)),
            pl.BlockSpec((H, D), lambda b, t: (0, 0)),
        ],
        out_specs=pl.BlockSpec((1, H, Tt), lambda b, t: (b, 0, t)),
        out_shape=jax.ShapeDtypeStruct((B, H, T), jnp.float32),
    )(x, W)

    R = B * H
    mesh = plsc.VectorSubcoreMesh(core_axis_name="c", subcore_axis_name="s")
    topk = functools.partial(
        pl.kernel,
        mesh=mesh,
        out_type=[
            jax.ShapeDtypeStruct((R * _K,), jnp.int32),
            jax.ShapeDtypeStruct((R * _K,), jnp.float32),
        ],
        scratch_types=[
            pltpu.VMEM((_RPW * _T,), jnp.float32),
            pltpu.VMEM((_RPW * _NCHUNK,), jnp.float32),
            pltpu.VMEM((_RPW * _K,), jnp.int32),
            pltpu.VMEM((_RPW * _K,), jnp.float32),
        ],
        compiler_params=pltpu.CompilerParams(
            needs_layout_passes=False, use_tc_tiling_on_sc=False),
    )(_sc_topk_body)

    idx, val = topk(scores.reshape(R * T))
    return idx.reshape(B, H, _K), val.reshape(B, H, _K)


# speedup vs baseline: 3.1443x; 1.0712x over previous
import functools

import jax
import jax.numpy as jnp
from jax import lax
from jax.experimental import pallas as pl
from jax.experimental.pallas import tpu as pltpu
from jax.experimental.pallas import tpu_sc as plsc

_K = 64
_L = 16
_RPW = 4
_T = 4096
_NCHUNK = _T // _L
_NSUPER = _NCHUNK // _L
_NEG_INF = float("-inf")


def _matmul_body(x_ref, w_ref, out_ref):
    out_ref[0] = jax.lax.dot_general(
        w_ref[...], x_ref[0],
        dimension_numbers=(((1,), (1,)), ((), ())),
        preferred_element_type=jnp.float32,
    )


def _sc_topk_body(scores_hbm, idx_hbm, val_hbm, rows_v, l1_v, oidx_v, oval_v):
    wid = lax.axis_index("s") * 2 + lax.axis_index("c")
    base = wid * _RPW
    pltpu.sync_copy(scores_hbm.at[pl.ds(base * _T, _RPW * _T)], rows_v)

    iota = lax.iota(jnp.int32, _L)

    def build(s, l2s):
        out = []
        for r in range(_RPW):
            goff = r * _T + s * (_L * _L) + iota * _L
            acc = plsc.load_gather(rows_v, [goff])
            for j in range(1, _L):
                acc = jnp.maximum(acc, plsc.load_gather(rows_v, [goff + j]))
            l1_v[pl.ds(r * _NCHUNK + s * _L, _L)] = acc
            sm = lax.reduce_max(acc, axes=(0,))
            out.append(jnp.where(iota == s, sm, l2s[r]))
        return tuple(out)

    l2s = tuple(jnp.full((_L,), _NEG_INF, jnp.float32) for _ in range(_RPW))
    l2s = lax.fori_loop(0, _NSUPER, build, l2s)

    R_ = range(_RPW)

    def extract(k, l2s):
        ms = [lax.reduce_max(l2s[r], axes=(0,)) for r in R_]
        ss = [plsc.all_reduce_ffs(l2s[r] == ms[r])[0] for r in R_]
        l1cs = [l1_v[pl.ds(r * _NCHUNK + ss[r] * _L, _L)] for r in R_]
        cs = [ss[r] * _L + plsc.all_reduce_ffs(l1cs[r] == ms[r])[0]
              for r in R_]
        dvs = [rows_v[pl.ds(r * _T + cs[r] * _L, _L)] for r in R_]
        lxs = [plsc.all_reduce_ffs(dvs[r] == ms[r])[0] for r in R_]
        kq, kr = k // _L, k % _L
        for r in R_:
            ob = r * _K + kq * _L
            oi = oidx_v[pl.ds(ob, _L)]
            oidx_v[pl.ds(ob, _L)] = jnp.where(
                iota == kr, cs[r] * _L + lxs[r], oi)
            ov = oval_v[pl.ds(ob, _L)]
            oval_v[pl.ds(ob, _L)] = jnp.where(iota == kr, ms[r], ov)
        dv2s = [jnp.where(iota == lxs[r], _NEG_INF, dvs[r]) for r in R_]
        for r in R_:
            rows_v[pl.ds(r * _T + cs[r] * _L, _L)] = dv2s[r]
        nms = [lax.reduce_max(dv2s[r], axes=(0,)) for r in R_]
        l1c2s = [jnp.where(iota == cs[r] - ss[r] * _L, nms[r],
                           l1_v[pl.ds(r * _NCHUNK + ss[r] * _L, _L)])
                 for r in R_]
        for r in R_:
            l1_v[pl.ds(r * _NCHUNK + ss[r] * _L, _L)] = l1c2s[r]
        sms = [lax.reduce_max(l1c2s[r], axes=(0,)) for r in R_]
        return tuple(jnp.where(iota == ss[r], sms[r], l2s[r]) for r in R_)

    lax.fori_loop(0, _K, extract, l2s)

    pltpu.sync_copy(oidx_v, idx_hbm.at[pl.ds(base * _K, _RPW * _K)])
    pltpu.sync_copy(oval_v, val_hbm.at[pl.ds(base * _K, _RPW * _K)])


def kernel(x, W):
    B, T, D = x.shape
    H = W.shape[0]
    Tt = 512

    scores = pl.pallas_call(
        _matmul_body,
        grid=(B, T // Tt),
        in_specs=[
            pl.BlockSpec((1, Tt, D), lambda b, t: (b, t, 0)),
            pl.BlockSpec((H, D), lambda b, t: (0, 0)),
        ],
        out_specs=pl.BlockSpec((1, H, Tt), lambda b, t: (b, 0, t)),
        out_shape=jax.ShapeDtypeStruct((B, H, T), jnp.float32),
    )(x, W)

    R = B * H
    mesh = plsc.VectorSubcoreMesh(core_axis_name="c", subcore_axis_name="s")
    topk = functools.partial(
        pl.kernel,
        mesh=mesh,
        out_type=[
            jax.ShapeDtypeStruct((R * _K,), jnp.int32),
            jax.ShapeDtypeStruct((R * _K,), jnp.float32),
        ],
        scratch_types=[
            pltpu.VMEM((_RPW * _T,), jnp.float32),
            pltpu.VMEM((_RPW * _NCHUNK,), jnp.float32),
            pltpu.VMEM((_RPW * _K,), jnp.int32),
            pltpu.VMEM((_RPW * _K,), jnp.float32),
        ],
        compiler_params=pltpu.CompilerParams(
            needs_layout_passes=False, use_tc_tiling_on_sc=False),
    )(_sc_topk_body)

    idx, val = topk(scores.reshape(R * T))
    return idx.reshape(B, H, _K), val.reshape(B, H, _K)


# speedup vs baseline: 3.4584x; 1.0999x over previous
import functools

import jax
import jax.numpy as jnp
from jax import lax
from jax.experimental import pallas as pl
from jax.experimental.pallas import tpu as pltpu
from jax.experimental.pallas import tpu_sc as plsc

_K = 64
_L = 16
_RPW = 4
_T = 4096
_NCHUNK = _T // _L
_NSUPER = _NCHUNK // _L
_NEG_INF = float("-inf")


def _matmul_body(x_ref, w_ref, out_ref):
    out_ref[0] = jax.lax.dot_general(
        w_ref[...], x_ref[0],
        dimension_numbers=(((1,), (1,)), ((), ())),
        preferred_element_type=jnp.float32,
    )


def _sc_topk_body(scores_hbm, idx_hbm, val_hbm, rows_v, l1_v, oidx_v, oval_v):
    wid = lax.axis_index("s") * 2 + lax.axis_index("c")
    base = wid * _RPW
    pltpu.sync_copy(scores_hbm.at[pl.ds(base * _T, _RPW * _T)], rows_v)

    iota = lax.iota(jnp.int32, _L)

    def build(s, l2s):
        out = []
        for r in range(_RPW):
            goff = r * _T + s * (_L * _L) + iota * _L
            acc = plsc.load_gather(rows_v, [goff])
            for j in range(1, _L):
                acc = jnp.maximum(acc, plsc.load_gather(rows_v, [goff + j]))
            l1_v[pl.ds(r * _NCHUNK + s * _L, _L)] = acc
            sm = lax.reduce_max(acc, axes=(0,))
            out.append(jnp.where(iota == s, sm, l2s[r]))
        return tuple(out)

    l2s = tuple(jnp.full((_L,), _NEG_INF, jnp.float32) for _ in range(_RPW))
    l2s = lax.fori_loop(0, _NSUPER, build, l2s)

    R_ = range(_RPW)

    def extract(k, l2s):
        ms = [lax.reduce_max(l2s[r], axes=(0,)) for r in R_]
        ss = [plsc.all_reduce_ffs(l2s[r] == ms[r])[0] for r in R_]
        l1cs = [l1_v[pl.ds(r * _NCHUNK + ss[r] * _L, _L)] for r in R_]
        cs = [ss[r] * _L + plsc.all_reduce_ffs(l1cs[r] == ms[r])[0]
              for r in R_]
        dvs = [rows_v[pl.ds(r * _T + cs[r] * _L, _L)] for r in R_]
        lxs = [plsc.all_reduce_ffs(dvs[r] == ms[r])[0] for r in R_]
        kq, kr = k // _L, k % _L
        for r in R_:
            ob = r * _K + kq * _L
            oi = oidx_v[pl.ds(ob, _L)]
            oidx_v[pl.ds(ob, _L)] = jnp.where(
                iota == kr, cs[r] * _L + lxs[r], oi)
            ov = oval_v[pl.ds(ob, _L)]
            oval_v[pl.ds(ob, _L)] = jnp.where(iota == kr, ms[r], ov)
        dv2s = [jnp.where(iota == lxs[r], _NEG_INF, dvs[r]) for r in R_]
        for r in R_:
            rows_v[pl.ds(r * _T + cs[r] * _L, _L)] = dv2s[r]
        nms = [lax.reduce_max(dv2s[r], axes=(0,)) for r in R_]
        l1c2s = [jnp.where(iota == cs[r] - ss[r] * _L, nms[r],
                           l1_v[pl.ds(r * _NCHUNK + ss[r] * _L, _L)])
                 for r in R_]
        for r in R_:
            l1_v[pl.ds(r * _NCHUNK + ss[r] * _L, _L)] = l1c2s[r]
        sms = [lax.reduce_max(l1c2s[r], axes=(0,)) for r in R_]
        return tuple(jnp.where(iota == ss[r], sms[r], l2s[r]) for r in R_)

    lax.fori_loop(0, _K, extract, l2s)

    pltpu.sync_copy(oidx_v, idx_hbm.at[pl.ds(base * _K, _RPW * _K)])
    pltpu.sync_copy(oval_v, val_hbm.at[pl.ds(base * _K, _RPW * _K)])


def kernel(x, W):
    B, T, D = x.shape
    H = W.shape[0]
    Tt = 1024

    scores = pl.pallas_call(
        _matmul_body,
        grid=(B, T // Tt),
        in_specs=[
            pl.BlockSpec((1, Tt, D), lambda b, t: (b, t, 0)),
            pl.BlockSpec((H, D), lambda b, t: (0, 0)),
        ],
        out_specs=pl.BlockSpec((1, H, Tt), lambda b, t: (b, 0, t)),
        out_shape=jax.ShapeDtypeStruct((B, H, T), jnp.float32),
    )(x, W)

    R = B * H
    mesh = plsc.VectorSubcoreMesh(core_axis_name="c", subcore_axis_name="s")
    topk = functools.partial(
        pl.kernel,
        mesh=mesh,
        out_type=[
            jax.ShapeDtypeStruct((R * _K,), jnp.int32),
            jax.ShapeDtypeStruct((R * _K,), jnp.float32),
        ],
        scratch_types=[
            pltpu.VMEM((_RPW * _T,), jnp.float32),
            pltpu.VMEM((_RPW * _NCHUNK,), jnp.float32),
            pltpu.VMEM((_RPW * _K,), jnp.int32),
            pltpu.VMEM((_RPW * _K,), jnp.float32),
        ],
        compiler_params=pltpu.CompilerParams(
            needs_layout_passes=False, use_tc_tiling_on_sc=False),
    )(_sc_topk_body)

    idx, val = topk(scores.reshape(R * T))
    return idx.reshape(B, H, _K), val.reshape(B, H, _K)
